# baseline (speedup 1.0000x reference)
import functools

import numpy as np
import jax
import jax.numpy as jnp
from jax import lax
from jax.experimental import pallas as pl
from jax.experimental.pallas import tpu as pltpu
from jax.experimental.pallas import tpu_sc as plsc

B = 1024
NBAND = 4
BAND = B // NBAND
N = 100000
K = 10
NC = 2
NS = 16
NW = NC * NS
ROWS_PW = BAND // NW
RB = 8
NBLK = ROWS_PW // RB
CHUNK = 3200
NCH = 31
TAIL = N - NCH * CHUNK
GROUP = 400
NVREG = GROUP // 16
NG_REG = CHUNK // GROUP
NG_TAIL = TAIL // GROUP
SURV = GROUP + 16

_W = np.zeros(16, np.float32)
_W[:K] = (1.0 / np.log2(np.arange(1, K + 1, dtype=np.float64) + 1.0)).astype(
    np.float32)


def _ndcg_rows(predictions, relevance):
  mesh = plsc.VectorSubcoreMesh(
      core_axis_name="c", subcore_axis_name="s", num_cores=NC,
      num_subcores=NS)

  @functools.partial(
      pl.kernel,
      out_type=jax.ShapeDtypeStruct((BAND,), jnp.float32),
      mesh=mesh,
      compiler_params=pltpu.CompilerParams(needs_layout_passes=False),
      scratch_types=[
          pltpu.VMEM((2 * RB, CHUNK), jnp.float32),
          pltpu.VMEM((2 * RB, CHUNK), jnp.float32),
          pltpu.VMEM((RB, TAIL), jnp.float32),
          pltpu.VMEM((RB, TAIL), jnp.float32),
          pltpu.VMEM((RB * 16,), jnp.float32),
          pltpu.VMEM((RB * 16,), jnp.float32),
          pltpu.VMEM((RB * 16,), jnp.float32),
          pltpu.VMEM((RB * 16,), jnp.float32),
          pltpu.VMEM((RB * 16,), jnp.float32),
          pltpu.VMEM((SURV,), jnp.float32),
          pltpu.VMEM((SURV,), jnp.float32),
          pltpu.VMEM((ROWS_PW,), jnp.float32),
          pltpu.SemaphoreType.DMA,
          pltpu.SemaphoreType.DMA,
      ],
  )
  def ndcg_kernel(pred_hbm, rel_hbm, out_hbm, pred_buf, rel_buf, pred_tl,
                  rel_tl, st_pv, st_pr, st_tp, st_rv, st_tr, sv_buf, sp_buf,
                  ndcg_buf, psem, rsem):
    wid = lax.axis_index("s") * NC + lax.axis_index("c")
    lane = lax.iota(jnp.int32, 16)
    w_vec = jnp.zeros((16,), jnp.float32)
    for k in range(K):
      w_vec = jnp.where(lane == k, float(_W[k]), w_vec)
    neg_inf = jnp.full((16,), -jnp.inf, jnp.float32)
    pos_inf = jnp.full((16,), jnp.inf, jnp.float32)
    row0 = wid * ROWS_PW

    def fetch_reg(b, c, parity):
      r8 = row0 + b * RB
      pltpu.async_copy(pred_hbm.at[pl.ds(r8, RB), pl.ds(c * CHUNK, CHUNK)],
                       pred_buf.at[pl.ds(parity * RB, RB)], psem)
      pltpu.async_copy(rel_hbm.at[pl.ds(r8, RB), pl.ds(c * CHUNK, CHUNK)],
                       rel_buf.at[pl.ds(parity * RB, RB)], rsem)

    def wait_reg(b, c, parity):
      r8 = row0 + b * RB
      pltpu.make_async_copy(
          pred_hbm.at[pl.ds(r8, RB), pl.ds(c * CHUNK, CHUNK)],
          pred_buf.at[pl.ds(parity * RB, RB)], psem).wait()
      pltpu.make_async_copy(
          rel_hbm.at[pl.ds(r8, RB), pl.ds(c * CHUNK, CHUNK)],
          rel_buf.at[pl.ds(parity * RB, RB)], rsem).wait()

    def fetch_tail(b):
      r8 = row0 + b * RB
      pltpu.async_copy(pred_hbm.at[pl.ds(r8, RB), pl.ds(NCH * CHUNK, TAIL)],
                       pred_tl, psem)
      pltpu.async_copy(rel_hbm.at[pl.ds(r8, RB), pl.ds(NCH * CHUNK, TAIL)],
                       rel_tl, rsem)

    def wait_tail(b):
      r8 = row0 + b * RB
      pltpu.make_async_copy(
          pred_hbm.at[pl.ds(r8, RB), pl.ds(NCH * CHUNK, TAIL)],
          pred_tl, psem).wait()
      pltpu.make_async_copy(
          rel_hbm.at[pl.ds(r8, RB), pl.ds(NCH * CHUNK, TAIL)],
          rel_tl, rsem).wait()

    def new_threshold(nv):
      t = jnp.min(jnp.where(lane < K, nv, pos_inf))
      return jnp.broadcast_to(t, (16,))

    def merge_pred_sorted(so, sv, sp):
      rsv = lax.rev(sv, (0,))
      rsp = lax.rev(sp, (0,))
      cur_v = st_pv[pl.ds(so, 16)]
      cur_p = st_pr[pl.ds(so, 16)]
      take = rsv > cur_v
      nv = jnp.where(take, rsv, cur_v)
      np_ = jnp.where(take, rsp, cur_p)
      nv, np_ = plsc.sort_key_val(nv, np_, descending=True)
      st_pv[pl.ds(so, 16)] = nv
      st_pr[pl.ds(so, 16)] = np_
      st_tp[pl.ds(so, 16)] = jnp.broadcast_to(nv[K - 1], (16,))

    def merge_rel_sorted(so, sv):
      rsv = lax.rev(sv, (0,))
      cur = st_rv[pl.ds(so, 16)]
      take = rsv > cur
      nv = jnp.where(take, rsv, cur)
      nv, _ = plsc.sort_key_val(nv, nv, descending=True)
      st_rv[pl.ds(so, 16)] = nv
      st_tr[pl.ds(so, 16)] = jnp.broadcast_to(nv[K - 1], (16,))

    def scan_chunk(pbuf, rbuf, prow0, ngroups):
      def row_scan(r, carry):
        so = pl.multiple_of(r * 16, 16)
        prow = prow0 + r

        def locate(buf, tvec, gb):
          cnt_a = jnp.zeros((16,), jnp.int32)
          cnt_b = jnp.zeros((16,), jnp.int32)
          for j in range(NVREG):
            v = buf[prow, pl.ds(gb + 16 * j, 16)]
            c = plsc.all_reduce_population_count(v > tvec)
            if j < 16:
              cnt_a = jnp.where(lane == j, c, cnt_a)
            else:
              cnt_b = jnp.where(lane == j - 16, c, cnt_b)
          return cnt_a, cnt_b

        def hot_cond(st):
          ma, mb = st
          return jnp.any(ma > 0) | jnp.any(mb > 0)

        def next_hot(ma, mb):
          anyA = jnp.any(ma > 0)
          ja = plsc.all_reduce_ffs(ma > 0)[0]
          jb = plsc.all_reduce_ffs(mb > 0)[0]
          j = jnp.where(anyA, ja, jb + 16)
          ma2 = jnp.where(lane == j, 0, ma)
          mb2 = jnp.where(lane == j - 16, 0, mb)
          return j, ma2, mb2

        def trigger_pred(tvec, gb):
          counts = locate(pbuf, tvec, gb)

          def body(st):
            ma, mb = st
            j, ma2, mb2 = next_hot(ma, mb)
            v = pbuf[prow, pl.ds(gb + 16 * j, 16)]
            pv = rbuf[prow, pl.ds(gb + 16 * j, 16)]
            m = v > st_tp[pl.ds(so, 16)]
            cnt = plsc.all_reduce_population_count(m)[0]
            plsc.store_compressed(sv_buf.at[pl.ds(0, 16)], v, mask=m)
            plsc.store_compressed(sp_buf.at[pl.ds(0, 16)], pv, mask=m)
            sv = sv_buf[pl.ds(0, 16)]
            sp = sp_buf[pl.ds(0, 16)]
            sv_buf[pl.ds(0, 16)] = neg_inf
            sv, sp = lax.cond(
                cnt > 1,
                lambda: tuple(plsc.sort_key_val(sv, sp, descending=True)),
                lambda: (sv, sp))

            @pl.when(cnt > 0)
            def _():
              merge_pred_sorted(so, sv, sp)

            return ma2, mb2

          lax.while_loop(hot_cond, body, counts)

        def trigger_rel(tvec, gb):
          counts = locate(rbuf, tvec, gb)

          def body(st):
            ma, mb = st
            j, ma2, mb2 = next_hot(ma, mb)
            v = rbuf[prow, pl.ds(gb + 16 * j, 16)]
            m = v > st_tr[pl.ds(so, 16)]
            cnt = plsc.all_reduce_population_count(m)[0]
            plsc.store_compressed(sv_buf.at[pl.ds(0, 16)], v, mask=m)
            sv = sv_buf[pl.ds(0, 16)]
            sv_buf[pl.ds(0, 16)] = neg_inf
            sv = lax.cond(
                cnt > 1,
                lambda: plsc.sort_key_val(sv, sv, descending=True)[0],
                lambda: sv)

            @pl.when(cnt > 0)
            def _():
              merge_rel_sorted(so, sv)

            return ma2, mb2

          lax.while_loop(hot_cond, body, counts)

        def group_body(g, c3):
          gb = pl.multiple_of(g * GROUP, GROUP)
          tp_vec = st_tp[pl.ds(so, 16)]
          tr_vec = st_tr[pl.ds(so, 16)]
          acc = [None] * 4
          for j in range(NVREG):
            v = pbuf[prow, pl.ds(gb + 16 * j, 16)]
            a = j % 4
            acc[a] = v if acc[a] is None else jnp.maximum(acc[a], v)
          pmax = jnp.maximum(jnp.maximum(acc[0], acc[1]),
                             jnp.maximum(acc[2], acc[3]))
          racc = [None] * 4
          for j in range(NVREG):
            v = rbuf[prow, pl.ds(gb + 16 * j, 16)]
            a = j % 4
            racc[a] = v if racc[a] is None else jnp.maximum(racc[a], v)
          rmax = jnp.maximum(jnp.maximum(racc[0], racc[1]),
                             jnp.maximum(racc[2], racc[3]))
          pcnt = plsc.all_reduce_population_count(pmax > tp_vec)[0]
          rcnt = plsc.all_reduce_population_count(rmax > tr_vec)[0]

          @pl.when(pcnt + rcnt > 0)
          def _():
            @pl.when(pcnt > 0)
            def _():
              trigger_pred(tp_vec, gb)

            @pl.when(rcnt > 0)
            def _():
              trigger_rel(tr_vec, gb)

          return c3

        return lax.fori_loop(0, ngroups, group_body, carry)

      lax.fori_loop(0, RB, row_scan, 0)

    for i in range(SURV // 16):
      sv_buf[pl.ds(16 * i, 16)] = neg_inf
    fetch_reg(0, 0, 0)

    def block_body(b, carry):
      for i in range(RB):
        so = pl.multiple_of(i * 16, 16)
        st_pv[pl.ds(so, 16)] = neg_inf
        st_pr[pl.ds(so, 16)] = jnp.zeros((16,), jnp.float32)
        st_tp[pl.ds(so, 16)] = neg_inf
        st_rv[pl.ds(so, 16)] = neg_inf
        st_tr[pl.ds(so, 16)] = neg_inf

      def chunk_body(c, c2):
        parity = c % 2
        wait_reg(b, c, parity)

        @pl.when(c < NCH - 1)
        def _():
          fetch_reg(b, c + 1, 1 - parity)

        @pl.when(c == NCH - 1)
        def _():
          fetch_tail(b)

        scan_chunk(pred_buf, rel_buf, parity * RB, NG_REG)
        return c2

      lax.fori_loop(0, NCH, chunk_body, 0)

      wait_tail(b)

      @pl.when(b < NBLK - 1)
      def _():
        fetch_reg(b + 1, 0, 0)

      scan_chunk(pred_tl, rel_tl, 0, NG_TAIL)

      def finalize(r, c4):
        so = pl.multiple_of(r * 16, 16)
        dcg = jnp.sum(jnp.where(lane < K, st_pr[pl.ds(so, 16)] * w_vec, 0.0))
        idcg = jnp.sum(jnp.where(lane < K, st_rv[pl.ds(so, 16)] * w_vec, 0.0))
        ndcg_v = jnp.broadcast_to(dcg, (16,)) / (
            jnp.broadcast_to(idcg, (16,)) + 1e-8)
        plsc.store_scatter(
            ndcg_buf,
            [jnp.broadcast_to(b * RB + r, (16,)).astype(jnp.int32)], ndcg_v,
            mask=lane == 0)
        return c4

      lax.fori_loop(0, RB, finalize, 0)
      return carry

    lax.fori_loop(0, NBLK, block_body, 0)
    pltpu.sync_copy(ndcg_buf, out_hbm.at[pl.ds(row0, ROWS_PW)])

  return ndcg_kernel(predictions, relevance)


def kernel(predictions, relevance_scores):
  parts = []
  for i in range(NBAND):
    p = lax.slice(predictions, (i * BAND, 0), ((i + 1) * BAND, N))
    r = lax.slice(relevance_scores, (i * BAND, 0), ((i + 1) * BAND, N))
    parts.append(_ndcg_rows(p, r))
  ndcg = jnp.concatenate(parts)
  return 1.0 - jnp.mean(ndcg)


# speedup vs baseline: 1.0295x; 1.0295x over previous
import functools

import numpy as np
import jax
import jax.numpy as jnp
from jax import lax
from jax.experimental import pallas as pl
from jax.experimental.pallas import tpu as pltpu
from jax.experimental.pallas import tpu_sc as plsc

B = 1024
N = 100000
K = 10
NC = 2
NS = 16
NW = NC * NS
ROWS_PW = B // NW
RB = 8
NBLK = ROWS_PW // RB
CHUNK = 3200
NCH = 31
TAIL = N - NCH * CHUNK
GROUP = 400
NVREG = GROUP // 16
NG_REG = CHUNK // GROUP
NG_TAIL = TAIL // GROUP
SURV = GROUP + 16

_W = np.zeros(16, np.float32)
_W[:K] = (1.0 / np.log2(np.arange(1, K + 1, dtype=np.float64) + 1.0)).astype(
    np.float32)


def _ndcg_rows(predictions, relevance):
  mesh = plsc.VectorSubcoreMesh(
      core_axis_name="c", subcore_axis_name="s", num_cores=NC,
      num_subcores=NS)

  @functools.partial(
      pl.kernel,
      out_type=jax.ShapeDtypeStruct((B,), jnp.float32),
      mesh=mesh,
      compiler_params=pltpu.CompilerParams(
          needs_layout_passes=False, allow_input_fusion=[True, True]),
      scratch_types=[
          pltpu.VMEM((2 * RB, CHUNK), jnp.float32),
          pltpu.VMEM((2 * RB, CHUNK), jnp.float32),
          pltpu.VMEM((RB, TAIL), jnp.float32),
          pltpu.VMEM((RB, TAIL), jnp.float32),
          pltpu.VMEM((RB * 16,), jnp.float32),
          pltpu.VMEM((RB * 16,), jnp.float32),
          pltpu.VMEM((RB * 16,), jnp.float32),
          pltpu.VMEM((RB * 16,), jnp.float32),
          pltpu.VMEM((RB * 16,), jnp.float32),
          pltpu.VMEM((SURV,), jnp.float32),
          pltpu.VMEM((SURV,), jnp.float32),
          pltpu.VMEM((ROWS_PW,), jnp.float32),
          pltpu.SemaphoreType.DMA,
          pltpu.SemaphoreType.DMA,
      ],
  )
  def ndcg_kernel(pred_hbm, rel_hbm, out_hbm, pred_buf, rel_buf, pred_tl,
                  rel_tl, st_pv, st_pr, st_tp, st_rv, st_tr, sv_buf, sp_buf,
                  ndcg_buf, psem, rsem):
    wid = lax.axis_index("s") * NC + lax.axis_index("c")
    lane = lax.iota(jnp.int32, 16)
    w_vec = jnp.zeros((16,), jnp.float32)
    for k in range(K):
      w_vec = jnp.where(lane == k, float(_W[k]), w_vec)
    neg_inf = jnp.full((16,), -jnp.inf, jnp.float32)
    pos_inf = jnp.full((16,), jnp.inf, jnp.float32)
    row0 = wid * ROWS_PW

    def fetch_reg(b, c, parity):
      r8 = row0 + b * RB
      pltpu.async_copy(pred_hbm.at[pl.ds(r8, RB), pl.ds(c * CHUNK, CHUNK)],
                       pred_buf.at[pl.ds(parity * RB, RB)], psem)
      pltpu.async_copy(rel_hbm.at[pl.ds(r8, RB), pl.ds(c * CHUNK, CHUNK)],
                       rel_buf.at[pl.ds(parity * RB, RB)], rsem)

    def wait_reg(b, c, parity):
      r8 = row0 + b * RB
      pltpu.make_async_copy(
          pred_hbm.at[pl.ds(r8, RB), pl.ds(c * CHUNK, CHUNK)],
          pred_buf.at[pl.ds(parity * RB, RB)], psem).wait()
      pltpu.make_async_copy(
          rel_hbm.at[pl.ds(r8, RB), pl.ds(c * CHUNK, CHUNK)],
          rel_buf.at[pl.ds(parity * RB, RB)], rsem).wait()

    def fetch_tail(b):
      r8 = row0 + b * RB
      pltpu.async_copy(pred_hbm.at[pl.ds(r8, RB), pl.ds(NCH * CHUNK, TAIL)],
                       pred_tl, psem)
      pltpu.async_copy(rel_hbm.at[pl.ds(r8, RB), pl.ds(NCH * CHUNK, TAIL)],
                       rel_tl, rsem)

    def wait_tail(b):
      r8 = row0 + b * RB
      pltpu.make_async_copy(
          pred_hbm.at[pl.ds(r8, RB), pl.ds(NCH * CHUNK, TAIL)],
          pred_tl, psem).wait()
      pltpu.make_async_copy(
          rel_hbm.at[pl.ds(r8, RB), pl.ds(NCH * CHUNK, TAIL)],
          rel_tl, rsem).wait()

    def new_threshold(nv):
      t = jnp.min(jnp.where(lane < K, nv, pos_inf))
      return jnp.broadcast_to(t, (16,))

    def merge_pred_sorted(so, sv, sp):
      rsv = lax.rev(sv, (0,))
      rsp = lax.rev(sp, (0,))
      cur_v = st_pv[pl.ds(so, 16)]
      cur_p = st_pr[pl.ds(so, 16)]
      take = rsv > cur_v
      nv = jnp.where(take, rsv, cur_v)
      np_ = jnp.where(take, rsp, cur_p)
      nv, np_ = plsc.sort_key_val(nv, np_, descending=True)
      st_pv[pl.ds(so, 16)] = nv
      st_pr[pl.ds(so, 16)] = np_
      st_tp[pl.ds(so, 16)] = jnp.broadcast_to(nv[K - 1], (16,))

    def merge_rel_sorted(so, sv):
      rsv = lax.rev(sv, (0,))
      cur = st_rv[pl.ds(so, 16)]
      take = rsv > cur
      nv = jnp.where(take, rsv, cur)
      nv, _ = plsc.sort_key_val(nv, nv, descending=True)
      st_rv[pl.ds(so, 16)] = nv
      st_tr[pl.ds(so, 16)] = jnp.broadcast_to(nv[K - 1], (16,))

    def scan_chunk(pbuf, rbuf, prow0, ngroups):
      def row_scan(r, carry):
        so = pl.multiple_of(r * 16, 16)
        prow = prow0 + r

        def locate(buf, tvec, gb):
          cnt_a = jnp.zeros((16,), jnp.int32)
          cnt_b = jnp.zeros((16,), jnp.int32)
          for j in range(NVREG):
            v = buf[prow, pl.ds(gb + 16 * j, 16)]
            c = plsc.all_reduce_population_count(v > tvec)
            if j < 16:
              cnt_a = jnp.where(lane == j, c, cnt_a)
            else:
              cnt_b = jnp.where(lane == j - 16, c, cnt_b)
          return cnt_a, cnt_b

        def hot_cond(st):
          ma, mb = st
          return jnp.any(ma > 0) | jnp.any(mb > 0)

        def next_hot(ma, mb):
          anyA = jnp.any(ma > 0)
          ja = plsc.all_reduce_ffs(ma > 0)[0]
          jb = plsc.all_reduce_ffs(mb > 0)[0]
          j = jnp.where(anyA, ja, jb + 16)
          ma2 = jnp.where(lane == j, 0, ma)
          mb2 = jnp.where(lane == j - 16, 0, mb)
          return j, ma2, mb2

        def trigger_pred(tvec, gb):
          counts = locate(pbuf, tvec, gb)

          def body(st):
            ma, mb = st
            j, ma2, mb2 = next_hot(ma, mb)
            v = pbuf[prow, pl.ds(gb + 16 * j, 16)]
            pv = rbuf[prow, pl.ds(gb + 16 * j, 16)]
            m = v > st_tp[pl.ds(so, 16)]
            cnt = plsc.all_reduce_population_count(m)[0]
            plsc.store_compressed(sv_buf.at[pl.ds(0, 16)], v, mask=m)
            plsc.store_compressed(sp_buf.at[pl.ds(0, 16)], pv, mask=m)
            sv = sv_buf[pl.ds(0, 16)]
            sp = sp_buf[pl.ds(0, 16)]
            sv_buf[pl.ds(0, 16)] = neg_inf
            sv, sp = lax.cond(
                cnt > 1,
                lambda: tuple(plsc.sort_key_val(sv, sp, descending=True)),
                lambda: (sv, sp))

            @pl.when(cnt > 0)
            def _():
              merge_pred_sorted(so, sv, sp)

            return ma2, mb2

          lax.while_loop(hot_cond, body, counts)

        def trigger_rel(tvec, gb):
          counts = locate(rbuf, tvec, gb)

          def body(st):
            ma, mb = st
            j, ma2, mb2 = next_hot(ma, mb)
            v = rbuf[prow, pl.ds(gb + 16 * j, 16)]
            m = v > st_tr[pl.ds(so, 16)]
            cnt = plsc.all_reduce_population_count(m)[0]
            plsc.store_compressed(sv_buf.at[pl.ds(0, 16)], v, mask=m)
            sv = sv_buf[pl.ds(0, 16)]
            sv_buf[pl.ds(0, 16)] = neg_inf
            sv = lax.cond(
                cnt > 1,
                lambda: plsc.sort_key_val(sv, sv, descending=True)[0],
                lambda: sv)

            @pl.when(cnt > 0)
            def _():
              merge_rel_sorted(so, sv)

            return ma2, mb2

          lax.while_loop(hot_cond, body, counts)

        def group_body(g, c3):
          gb = pl.multiple_of(g * GROUP, GROUP)
          tp_vec = st_tp[pl.ds(so, 16)]
          tr_vec = st_tr[pl.ds(so, 16)]
          acc = [None] * 4
          for j in range(NVREG):
            v = pbuf[prow, pl.ds(gb + 16 * j, 16)]
            a = j % 4
            acc[a] = v if acc[a] is None else jnp.maximum(acc[a], v)
          pmax = jnp.maximum(jnp.maximum(acc[0], acc[1]),
                             jnp.maximum(acc[2], acc[3]))
          racc = [None] * 4
          for j in range(NVREG):
            v = rbuf[prow, pl.ds(gb + 16 * j, 16)]
            a = j % 4
            racc[a] = v if racc[a] is None else jnp.maximum(racc[a], v)
          rmax = jnp.maximum(jnp.maximum(racc[0], racc[1]),
                             jnp.maximum(racc[2], racc[3]))
          pcnt = plsc.all_reduce_population_count(pmax > tp_vec)[0]
          rcnt = plsc.all_reduce_population_count(rmax > tr_vec)[0]

          @pl.when(pcnt + rcnt > 0)
          def _():
            @pl.when(pcnt > 0)
            def _():
              trigger_pred(tp_vec, gb)

            @pl.when(rcnt > 0)
            def _():
              trigger_rel(tr_vec, gb)

          return c3

        return lax.fori_loop(0, ngroups, group_body, carry)

      lax.fori_loop(0, RB, row_scan, 0)

    for i in range(SURV // 16):
      sv_buf[pl.ds(16 * i, 16)] = neg_inf
    fetch_reg(0, 0, 0)

    def block_body(b, carry):
      for i in range(RB):
        so = pl.multiple_of(i * 16, 16)
        st_pv[pl.ds(so, 16)] = neg_inf
        st_pr[pl.ds(so, 16)] = jnp.zeros((16,), jnp.float32)
        st_tp[pl.ds(so, 16)] = neg_inf
        st_rv[pl.ds(so, 16)] = neg_inf
        st_tr[pl.ds(so, 16)] = neg_inf

      def chunk_body(c, c2):
        parity = c % 2
        wait_reg(b, c, parity)

        @pl.when(c < NCH - 1)
        def _():
          fetch_reg(b, c + 1, 1 - parity)

        @pl.when(c == NCH - 1)
        def _():
          fetch_tail(b)

        scan_chunk(pred_buf, rel_buf, parity * RB, NG_REG)
        return c2

      lax.fori_loop(0, NCH, chunk_body, 0)

      wait_tail(b)

      @pl.when(b < NBLK - 1)
      def _():
        fetch_reg(b + 1, 0, 0)

      scan_chunk(pred_tl, rel_tl, 0, NG_TAIL)

      def finalize(r, c4):
        so = pl.multiple_of(r * 16, 16)
        dcg = jnp.sum(jnp.where(lane < K, st_pr[pl.ds(so, 16)] * w_vec, 0.0))
        idcg = jnp.sum(jnp.where(lane < K, st_rv[pl.ds(so, 16)] * w_vec, 0.0))
        ndcg_v = jnp.broadcast_to(dcg, (16,)) / (
            jnp.broadcast_to(idcg, (16,)) + 1e-8)
        plsc.store_scatter(
            ndcg_buf,
            [jnp.broadcast_to(b * RB + r, (16,)).astype(jnp.int32)], ndcg_v,
            mask=lane == 0)
        return c4

      lax.fori_loop(0, RB, finalize, 0)
      return carry

    lax.fori_loop(0, NBLK, block_body, 0)
    pltpu.sync_copy(ndcg_buf, out_hbm.at[pl.ds(row0, ROWS_PW)])

  return ndcg_kernel(predictions, relevance)


def kernel(predictions, relevance_scores):
  ndcg = _ndcg_rows(predictions, relevance_scores)
  return 1.0 - jnp.mean(ndcg)
